# Initial kernel scaffold; baseline (speedup 1.0000x reference)
import jax
import jax.numpy as jnp
from jax.experimental import pallas as pl


def kernel(x, edge_index, W0_l, W0_r, b0, W1_l, W1_r, b1):
    raise NotImplementedError("write your pallas kernel here")


# speedup vs baseline: 3.3347x; 3.3347x over previous
import jax
import jax.numpy as jnp
from jax import lax
from jax.experimental import pallas as pl
from jax.experimental.pallas import tpu as pltpu
from jax.experimental.pallas import tpu_sc as plsc

N = 10000
E = 160000
D_IN = 256
HID = 512
LANES = 128
NC = 2
NS = 16
K = 80
EPT = E // NS
NBATCH = EPT // K
NP_ = 10240
RPT = NP_ // NS
BN = 1000


def _make_sc_agg(C, with_cnt):
    out_type = [jax.ShapeDtypeStruct((C * NP_, LANES), jnp.float32)]
    if with_cnt:
        out_type.append(jax.ShapeDtypeStruct((NP_,), jnp.float32))
    mesh = plsc.VectorSubcoreMesh(core_axis_name="c", subcore_axis_name="s",
                                  num_cores=NC, num_subcores=NS)
    scratch_types = [
        pltpu.VMEM((K,), jnp.int32),
        pltpu.VMEM((K,), jnp.int32),
        pltpu.VMEM((K, LANES), jnp.float32),
        pltpu.VMEM((K,), jnp.float32),
        pltpu.VMEM_SHARED((NP_, LANES), jnp.float32),
        pltpu.VMEM_SHARED((NP_,), jnp.float32),
        pltpu.SemaphoreType.DMA,
    ]

    def body(xs, src, dst, z2d, z1d, onesk, *rest):
        if with_cnt:
            agg_o, cnt_o = rest[0], rest[1]
            rest = rest[2:]
        else:
            agg_o = rest[0]
            rest = rest[1:]
        srcv, dstv, rows, onev, agg_sp, cnt_sp, sem = rest
        ci = lax.axis_index("c")
        si = lax.axis_index("s")
        r0 = si * RPT
        ebase = si * EPT
        if with_cnt:
            pltpu.sync_copy(onesk, onev)

        for cc in range(C // NC):
            chunk = cc * NC + ci
            pltpu.sync_copy(z2d.at[pl.ds(r0, RPT)], agg_sp.at[pl.ds(r0, RPT)])
            if with_cnt:
                @pl.when(jnp.logical_and(ci == 0, si == 0))
                def _():
                    pltpu.sync_copy(z1d, cnt_sp)
            plsc.subcore_barrier()

            def batch(i, _):
                e0 = pl.multiple_of(ebase + i * K, 8)
                pltpu.sync_copy(src.at[pl.ds(e0, K)], srcv)
                pltpu.sync_copy(dst.at[pl.ds(e0, K)], dstv)
                bias = chunk * N
                for j in range(K // 16):
                    srcv[pl.ds(j * 16, 16)] = srcv[pl.ds(j * 16, 16)] + bias
                pltpu.async_copy(xs.at[srcv], rows, sem).wait()
                pltpu.sync_copy(rows, agg_sp.at[dstv], add=True)
                if with_cnt:
                    @pl.when(ci == 0)
                    def _():
                        pltpu.sync_copy(onev, cnt_sp.at[dstv], add=True)
                return 0

            lax.fori_loop(0, NBATCH, batch, 0)
            plsc.subcore_barrier()

            o0 = chunk * NP_ + r0
            pltpu.sync_copy(agg_sp.at[pl.ds(r0, RPT)], agg_o.at[pl.ds(o0, RPT)])
            if with_cnt:
                @pl.when(jnp.logical_and(ci == 0, si == 0))
                def _():
                    pltpu.sync_copy(cnt_sp, cnt_o)
            if cc + 1 < C // NC:
                plsc.subcore_barrier()

    return pl.kernel(body, out_type=out_type, mesh=mesh,
                     scratch_types=scratch_types)


_sc_agg2 = _make_sc_agg(2, with_cnt=True)
_sc_agg4 = _make_sc_agg(4, with_cnt=False)


def _tc0_body(agg_ref, cnt_ref, x_ref, wl_ref, wr_ref, b_ref, out_ref):
    inv = 1.0 / jnp.maximum(cnt_ref[...], 1.0)
    wl = wl_ref[...]
    acc = jnp.dot(x_ref[...], wr_ref[...], preferred_element_type=jnp.float32)
    for c in range(D_IN // LANES):
        acc += jnp.dot(agg_ref[c] * inv, wl[c * LANES:(c + 1) * LANES, :],
                       preferred_element_type=jnp.float32)
    h = jnp.maximum(acc + b_ref[...], 0.0)
    for c in range(HID // LANES):
        out_ref[c] = h[:, c * LANES:(c + 1) * LANES]


_tc_layer0 = pl.pallas_call(
    _tc0_body,
    grid=(N // BN,),
    in_specs=[
        pl.BlockSpec((D_IN // LANES, BN, LANES), lambda i: (0, i, 0)),
        pl.BlockSpec((BN, 1), lambda i: (i, 0)),
        pl.BlockSpec((BN, D_IN), lambda i: (i, 0)),
        pl.BlockSpec((D_IN, HID), lambda i: (0, 0)),
        pl.BlockSpec((D_IN, HID), lambda i: (0, 0)),
        pl.BlockSpec((1, HID), lambda i: (0, 0)),
    ],
    out_specs=pl.BlockSpec((HID // LANES, BN, LANES), lambda i: (0, i, 0)),
    out_shape=jax.ShapeDtypeStruct((HID // LANES, N, LANES), jnp.float32),
)


def _tc1_body(agg_ref, cnt_ref, h_ref, wl_ref, wr_ref, b_ref, out_ref):
    inv = 1.0 / jnp.maximum(cnt_ref[...], 1.0)
    wl = wl_ref[...]
    wr = wr_ref[...]
    acc = jnp.broadcast_to(b_ref[...], (BN, HID))
    for c in range(HID // LANES):
        acc += jnp.dot(agg_ref[c] * inv, wl[c * LANES:(c + 1) * LANES, :],
                       preferred_element_type=jnp.float32)
        acc += jnp.dot(h_ref[c], wr[c * LANES:(c + 1) * LANES, :],
                       preferred_element_type=jnp.float32)
    out_ref[...] = jnp.maximum(acc, 0.0)


_tc_layer1 = pl.pallas_call(
    _tc1_body,
    grid=(N // BN,),
    in_specs=[
        pl.BlockSpec((HID // LANES, BN, LANES), lambda i: (0, i, 0)),
        pl.BlockSpec((BN, 1), lambda i: (i, 0)),
        pl.BlockSpec((HID // LANES, BN, LANES), lambda i: (0, i, 0)),
        pl.BlockSpec((HID, HID), lambda i: (0, 0)),
        pl.BlockSpec((HID, HID), lambda i: (0, 0)),
        pl.BlockSpec((1, HID), lambda i: (0, 0)),
    ],
    out_specs=pl.BlockSpec((BN, HID), lambda i: (i, 0)),
    out_shape=jax.ShapeDtypeStruct((N, HID), jnp.float32),
)


def kernel(x, edge_index, W0_l, W0_r, b0, W1_l, W1_r, b1):
    src = edge_index[0]
    dst = edge_index[1]
    xs0 = x.reshape(N, D_IN // LANES, LANES).transpose(1, 0, 2)
    xs0 = xs0.reshape((D_IN // LANES) * N, LANES)
    z2d = jnp.zeros((NP_, LANES), jnp.float32)
    z1d = jnp.zeros((NP_,), jnp.float32)
    onesk = jnp.ones((K,), jnp.float32)

    agg0, cnt = _sc_agg2(xs0, src, dst, z2d, z1d, onesk)
    cnt2 = cnt.reshape(NP_, 1)
    h_split = _tc_layer0(agg0.reshape(D_IN // LANES, NP_, LANES), cnt2, x,
                         W0_l, W0_r, b0.reshape(1, HID))
    agg1, = _sc_agg4(h_split.reshape((HID // LANES) * N, LANES), src, dst,
                     z2d, z1d, onesk)
    out = _tc_layer1(agg1.reshape(HID // LANES, NP_, LANES), cnt2, h_split,
                     W1_l, W1_r, b1.reshape(1, HID))
    return out
